# baseline (speedup 1.0000x reference)
import functools

import jax
import jax.numpy as jnp
from jax import lax
from jax.experimental import pallas as pl
from jax.experimental.pallas import tpu as pltpu
from jax.experimental.pallas import tpu_sc as plsc

_BOS_IDX = 1000001
_EOS_IDX = 1000002
_EMBED = 64
_NC = 2
_NS = 16
_NW = _NC * _NS
_LAG = 64


@functools.partial(jax.jit, static_argnums=(4, 5))
def _sc_gather(table, idx, seq, pos, per_w, seq_len):
    b_total = idx.shape[0] // seq_len
    n_blocks = per_w // 16
    mesh = plsc.VectorSubcoreMesh(core_axis_name="c", subcore_axis_name="s")

    @functools.partial(
        pl.kernel,
        mesh=mesh,
        out_type=jax.ShapeDtypeStruct((b_total, seq_len, _EMBED), jnp.float32),
        scratch_types=[
            pltpu.VMEM((per_w,), jnp.int32),
            pltpu.VMEM((per_w,), jnp.int32),
            pltpu.VMEM((per_w,), jnp.int32),
            pltpu.SemaphoreType.DMA,
        ],
    )
    def k(table_hbm, idx_hbm, seq_hbm, pos_hbm, out_hbm, idx_s, seq_s, pos_s, sem):
        wid = lax.axis_index("s") * _NC + lax.axis_index("c")
        base = wid * per_w
        pltpu.sync_copy(idx_hbm.at[pl.ds(base, per_w)], idx_s)
        pltpu.sync_copy(seq_hbm.at[pl.ds(base, per_w)], seq_s)
        pltpu.sync_copy(pos_hbm.at[pl.ds(base, per_w)], pos_s)

        def drain16():
            for _ in range(16):
                pltpu.make_async_copy(
                    table_hbm.at[0], out_hbm.at[0, 0], sem
                ).wait()

        def body(kblk, carry):
            ivec = idx_s[pl.ds(kblk * 16, 16)]
            svec = seq_s[pl.ds(kblk * 16, 16)]
            pvec = pos_s[pl.ds(kblk * 16, 16)]
            for j in range(16):
                pltpu.async_copy(
                    table_hbm.at[ivec[j]],
                    out_hbm.at[svec[j], pvec[j]],
                    sem,
                )

            @pl.when(kblk >= _LAG)
            def _():
                drain16()

            return carry

        lax.fori_loop(0, n_blocks, body, 0)
        for _ in range(_LAG):
            drain16()

    return k(table, idx, seq, pos)


def kernel(indices, table):
    B, L = indices.shape
    bos = jnp.full((B, 1), _BOS_IDX, jnp.int32)
    eos = jnp.full((B, 1), _EOS_IDX, jnp.int32)
    idx = jnp.concatenate([bos, indices.astype(jnp.int32), eos], axis=1)
    n_rows = B * (L + 2)
    per_w = n_rows // _NW
    t = jnp.arange(n_rows, dtype=jnp.int32)
    seq = t // (L + 2)
    pos = t - seq * (L + 2)
    out = _sc_gather(table, idx.reshape(-1), seq, pos, per_w, L + 2)
    return out


# speedup vs baseline: 2.5545x; 2.5545x over previous
import functools

import jax
import jax.numpy as jnp
from jax import lax
from jax.experimental import pallas as pl
from jax.experimental.pallas import tpu as pltpu
from jax.experimental.pallas import tpu_sc as plsc

_BOS_IDX = 1000001
_EOS_IDX = 1000002
_EMBED = 64
_NC = 2
_NS = 16
_NW = _NC * _NS
_CHUNK = 128
_K = 2
_NBUF = 2
_GROUP = _K * _CHUNK


@functools.partial(jax.jit, static_argnums=(2, 3))
def _sc_gather(table_pairs, idx_blocks, per_w, n_chunks):
    n_rows = _NW * per_w
    n_groups = n_chunks // _K
    mesh = plsc.VectorSubcoreMesh(core_axis_name="c", subcore_axis_name="s")

    @functools.partial(
        pl.kernel,
        mesh=mesh,
        out_type=jax.ShapeDtypeStruct((n_rows // 2, 2 * _EMBED), jnp.float32),
        scratch_types=[
            pltpu.VMEM((n_chunks, _CHUNK), jnp.int32),
            pltpu.VMEM((_NBUF, _K, _CHUNK), jnp.int32),
            pltpu.VMEM((_NBUF, _GROUP, 2 * _EMBED), jnp.float32),
            pltpu.VMEM((_GROUP // 2, 2 * _EMBED), jnp.float32),
            pltpu.SemaphoreType.DMA,
            pltpu.SemaphoreType.DMA,
        ],
    )
    def k(tp_hbm, idx_hbm, out_hbm, idx_v, pidx_v, rows_v, pk_v, sem0, sem1):
        wid = lax.axis_index("s") * _NC + lax.axis_index("c")
        base = wid * per_w
        sems = (sem0, sem1)
        pltpu.sync_copy(idx_hbm.at[wid], idx_v)

        def fire(g, b):
            cps = []
            for j in range(_K):
                c = g * _K + j
                for v in range(_CHUNK // 16):
                    pidx_v[b, j, pl.ds(v * 16, 16)] = (
                        jax.lax.shift_right_logical(
                            idx_v[c, pl.ds(v * 16, 16)], 1
                        )
                    )
                cps.append(pltpu.async_copy(
                    tp_hbm.at[pidx_v.at[b, j]],
                    rows_v.at[b, pl.ds(j * _CHUNK, _CHUNK)],
                    sems[b],
                ))
            return cps

        def extract(g, b):
            def body(vv, carry):
                c0 = g * _K
                j = vv >> 3
                v = vv & 7
                raw = idx_v[c0 + j, pl.ds(v * 16, 16)]
                for e in range(16):
                    h = raw[e] & 1
                    t = j * _CHUNK + v * 16 + e
                    tt = (j * _CHUNK + v * 16 + e) // 2 if False else (
                        j * (_CHUNK // 2) + v * 8 + (e // 2)
                    )
                    for q in range(_EMBED // 16):
                        pk_v[tt, pl.ds((e % 2) * _EMBED + q * 16, 16)] = (
                            rows_v[b, t, pl.ds(h * _EMBED + q * 16, 16)]
                        )
                return carry

            lax.fori_loop(0, _K * (_CHUNK // 16), body, 0)

        pending = [fire(0, 0), fire(1, 1)]
        for g in range(n_groups):
            b = g % _NBUF
            for cp in pending[b]:
                cp.wait()
            extract(g, b)
            valid = min(_GROUP // 2, (per_w - g * _GROUP) // 2)
            off = pl.multiple_of((base + g * _GROUP) // 2, 8)
            pltpu.sync_copy(
                pk_v.at[pl.ds(0, valid)],
                out_hbm.at[pl.ds(off, valid)],
            )
            ng = g + _NBUF
            if ng < n_groups:
                pending[b] = fire(ng, b)
            else:
                pending[b] = []

    return k(table_pairs, idx_blocks)


def kernel(indices, table):
    B, L = indices.shape
    bos = jnp.full((B, 1), _BOS_IDX, jnp.int32)
    eos = jnp.full((B, 1), _EOS_IDX, jnp.int32)
    idx = jnp.concatenate([bos, indices.astype(jnp.int32), eos], axis=1)
    n_rows = B * (L + 2)
    per_w = n_rows // _NW
    n_chunks = -(-per_w // (_CHUNK * _K)) * _K
    pad = n_chunks * _CHUNK - per_w
    idx_blocks = jnp.pad(idx.reshape(_NW, per_w), ((0, 0), (0, pad)))
    idx_blocks = idx_blocks.reshape(_NW, n_chunks, _CHUNK)
    table_pairs = jnp.concatenate(
        [table, jnp.zeros((1, _EMBED), jnp.float32)], axis=0
    ).reshape(-1, 2 * _EMBED)
    out = _sc_gather(table_pairs, idx_blocks, per_w, n_chunks)
    return out.reshape(B, L + 2, _EMBED)
